# Initial kernel scaffold; baseline (speedup 1.0000x reference)
import jax
import jax.numpy as jnp
from jax.experimental import pallas as pl


def kernel(x, edge_index, label, mask, W0, W1, W_att, b_att, u_att):
    raise NotImplementedError("write your pallas kernel here")


# speedup vs baseline: 3.4636x; 3.4636x over previous
import functools

import jax
import jax.numpy as jnp
from jax import lax
from jax.experimental import pallas as pl
from jax.experimental.pallas import tpu as pltpu
from jax.experimental.pallas import tpu_sc as plsc

N = 10000
D_IN = 256
NHID = 512
D_OUT = 128
NMETA = 4
NEDGE = 160000
NATT = 4
WDECAY = 0.0005

NC = 2
NT = 16
CH = 128
NCHUNK = 80
EPT = NCHUNK * CH
NACC = 10240
HALF = 128
RPT = NACC // NT


def _sc_pass1(x2flat, srcp2, dstp, zrow):
    mesh = plsc.VectorSubcoreMesh(core_axis_name="c", subcore_axis_name="s", num_cores=NC, num_subcores=NT)

    @functools.partial(
        pl.kernel,
        out_type=jax.ShapeDtypeStruct((NMETA, NC, NACC, HALF), jnp.float32),
        mesh=mesh,
        scratch_types=[
            pltpu.VMEM((NCHUNK, CH), jnp.int32),
            pltpu.VMEM((NCHUNK, CH), jnp.int32),
            pltpu.VMEM((CH, HALF), jnp.float32),
            pltpu.VMEM_SHARED((NACC, HALF), jnp.float32),
        ],
    )
    def k(x_hbm, src_hbm, dst_hbm, z_hbm, g_hbm, src_v, dst_v, rows_v, acc):
        c = lax.axis_index("c")
        s = lax.axis_index("s")
        for m in range(NMETA):
            pltpu.sync_copy(z_hbm, acc.at[pl.ds(s * RPT, RPT)])
            plsc.subcore_barrier()
            pltpu.sync_copy(src_hbm.at[c, m, s], src_v)
            pltpu.sync_copy(dst_hbm.at[m, s], dst_v)

            def chunk(j, carry):
                pltpu.sync_copy(x_hbm.at[src_v.at[j]], rows_v)
                pltpu.sync_copy(rows_v, acc.at[dst_v.at[j]], add=True)
                return carry

            lax.fori_loop(0, NCHUNK, chunk, 0)
            plsc.subcore_barrier()
            pltpu.sync_copy(acc.at[pl.ds(s * RPT, RPT)],
                            g_hbm.at[m, c, pl.ds(s * RPT, RPT)])
            plsc.subcore_barrier()

    return k(x2flat, srcp2, dstp, zrow)


def _sc_pass2(pre2flat, srcp3, dstp, zrow):
    mesh = plsc.VectorSubcoreMesh(core_axis_name="c", subcore_axis_name="s", num_cores=NC, num_subcores=NT)

    @functools.partial(
        pl.kernel,
        out_type=jax.ShapeDtypeStruct((NMETA, NACC, D_OUT), jnp.float32),
        mesh=mesh,
        scratch_types=[
            pltpu.VMEM((NCHUNK, CH), jnp.int32),
            pltpu.VMEM((NCHUNK, CH), jnp.int32),
            pltpu.VMEM((CH, D_OUT), jnp.float32),
            pltpu.VMEM_SHARED((NACC, D_OUT), jnp.float32),
        ],
    )
    def k(t_hbm, src_hbm, dst_hbm, z_hbm, h2_hbm, src_v, dst_v, rows_v, acc):
        c = lax.axis_index("c")
        s = lax.axis_index("s")
        for kk in range(NMETA // NC):
            m = c * (NMETA // NC) + kk
            pltpu.sync_copy(z_hbm, acc.at[pl.ds(s * RPT, RPT)])
            plsc.subcore_barrier()
            pltpu.sync_copy(src_hbm.at[m, s], src_v)
            pltpu.sync_copy(dst_hbm.at[m, s], dst_v)

            def chunk(j, carry):
                pltpu.sync_copy(t_hbm.at[src_v.at[j]], rows_v)
                pltpu.sync_copy(rows_v, acc.at[dst_v.at[j]], add=True)
                return carry

            lax.fori_loop(0, NCHUNK, chunk, 0)
            plsc.subcore_barrier()
            pltpu.sync_copy(acc.at[pl.ds(s * RPT, RPT)],
                            h2_hbm.at[m, pl.ds(s * RPT, RPT)])
            plsc.subcore_barrier()

    return k(pre2flat, srcp3, dstp, zrow)


BLK1 = 512


def _tc_gcn_body(g0, g1, w0lo, w0hi, w1, out):
    a = g0[0, 0]
    b = g1[0, 0]
    h = jnp.dot(a, w0lo[...], preferred_element_type=jnp.float32)
    h = h + jnp.dot(b, w0hi[...], preferred_element_type=jnp.float32)
    h = jnp.maximum(h, 0.0)
    nrm = jnp.sqrt(jnp.sum(h * h, axis=1, keepdims=True)) + 1e-12
    h = h / nrm
    out[0] = jnp.dot(h, w1[...], preferred_element_type=jnp.float32)


def _tc_gcn(g, W0, W1):
    nb = NACC // BLK1
    return pl.pallas_call(
        _tc_gcn_body,
        grid=(NMETA, nb),
        in_specs=[
            pl.BlockSpec((1, 1, BLK1, HALF), lambda m, j: (m, 0, j, 0)),
            pl.BlockSpec((1, 1, BLK1, HALF), lambda m, j: (m, 1, j, 0)),
            pl.BlockSpec((HALF, NHID), lambda m, j: (0, 0)),
            pl.BlockSpec((HALF, NHID), lambda m, j: (0, 0)),
            pl.BlockSpec((NHID, D_OUT), lambda m, j: (0, 0)),
        ],
        out_specs=pl.BlockSpec((1, BLK1, D_OUT), lambda m, j: (m, j, 0)),
        out_shape=jax.ShapeDtypeStruct((NMETA, NACC, D_OUT), jnp.float32),
    )(g, g, W0[:HALF], W0[HALF:], W1)


BLK2 = 512


def _tc_att_body(h2, wat, v_ref, wsq_ref):
    j = pl.program_id(0)

    @pl.when(j == 0)
    def _init():
        for m in range(NMETA):
            for a in range(NATT):
                v_ref[m, a] = 0.0
        wsq_ref[0, 0] = 0.0

    wsq_ref[0, 0] += jnp.sum(wat[...] * wat[...])
    for m in range(NMETA):
        hm = h2[m]
        for a in range(NATT):
            v_ref[m, a] += jnp.sum(hm * wat[a])


def _tc_att(h2, wat):
    nb = NACC // BLK2
    return pl.pallas_call(
        _tc_att_body,
        grid=(nb,),
        in_specs=[
            pl.BlockSpec((NMETA, BLK2, D_OUT), lambda j: (0, j, 0)),
            pl.BlockSpec((NATT, BLK2, D_OUT), lambda j: (0, j, 0)),
        ],
        out_specs=[
            pl.BlockSpec(memory_space=pltpu.SMEM),
            pl.BlockSpec(memory_space=pltpu.SMEM),
        ],
        out_shape=[
            jax.ShapeDtypeStruct((NMETA, NATT), jnp.float32),
            jax.ShapeDtypeStruct((1, 1), jnp.float32),
        ],
    )(h2, wat)


def _tc_final_body(h2, label, maskc, vpad, bpad, upad, w0, w1, wsq,
                   loss_ref, acc_ref, s_sm):
    j = pl.program_id(0)
    nb = pl.num_programs(0)

    @pl.when(j == 0)
    def _init():
        s_sm[0] = 0.0
        s_sm[1] = 0.0
        s_sm[2] = 0.0
        s_sm[3] = jnp.sum(w0[...] * w0[...]) + jnp.sum(w1[...] * w1[...])

    vt = jnp.tanh(vpad[...] + bpad[...])
    vu = jnp.sum(vt * upad[...], axis=1, keepdims=True)
    ridx = lax.broadcasted_iota(jnp.int32, (8, 1), 0)
    z = jnp.where(ridx < NMETA, vu, -1e30)
    ez = jnp.exp(z - jnp.max(z))
    alpha = ez / jnp.sum(ez)

    fb = jnp.zeros((BLK2, D_OUT), jnp.float32)
    for m in range(NMETA):
        fb = fb + alpha[m, 0] * h2[m]

    rmax = jnp.max(fb, axis=1, keepdims=True)
    ex = jnp.exp(fb - rmax)
    se = jnp.sum(ex, axis=1, keepdims=True)
    logp = fb - rmax - jnp.log(se)
    lb = label[...]
    mk = maskc[...]
    ce = -jnp.sum(lb * logp, axis=1, keepdims=True)

    cidx = lax.broadcasted_iota(jnp.int32, (BLK2, D_OUT), 1)
    am_f = jnp.min(jnp.where(fb == rmax, cidx, D_OUT), axis=1, keepdims=True)
    lmax = jnp.max(lb, axis=1, keepdims=True)
    am_l = jnp.min(jnp.where(lb == lmax, cidx, D_OUT), axis=1, keepdims=True)
    corr = jnp.where(am_f == am_l, 1.0, 0.0)

    s_sm[0] += jnp.sum(ce * mk)
    s_sm[1] += jnp.sum(mk)
    s_sm[2] += jnp.sum(corr * mk)

    @pl.when(j == nb - 1)
    def _fin():
        l2b = jnp.sum(bpad[...] * bpad[...])
        l2u = jnp.sum(upad[0:1, :] * upad[0:1, :])
        l2all = 0.5 * (s_sm[3] + wsq[0, 0] + l2b + l2u)
        loss_ref[0, 0] = WDECAY * l2all + s_sm[0] / s_sm[1]
        acc_ref[0, 0] = s_sm[2] / s_sm[1]


def _tc_final(h2, label_p, mask_p, vpad, bpad, upad, W0, W1, wsq):
    nb = NACC // BLK2
    return pl.pallas_call(
        _tc_final_body,
        grid=(nb,),
        in_specs=[
            pl.BlockSpec((NMETA, BLK2, D_OUT), lambda j: (0, j, 0)),
            pl.BlockSpec((BLK2, D_OUT), lambda j: (j, 0)),
            pl.BlockSpec((BLK2, 1), lambda j: (j, 0)),
            pl.BlockSpec((8, 128), lambda j: (0, 0)),
            pl.BlockSpec((8, 128), lambda j: (0, 0)),
            pl.BlockSpec((8, 128), lambda j: (0, 0)),
            pl.BlockSpec((D_IN, NHID), lambda j: (0, 0)),
            pl.BlockSpec((NHID, D_OUT), lambda j: (0, 0)),
            pl.BlockSpec(memory_space=pltpu.SMEM),
        ],
        out_specs=[
            pl.BlockSpec(memory_space=pltpu.SMEM),
            pl.BlockSpec(memory_space=pltpu.SMEM),
        ],
        out_shape=[
            jax.ShapeDtypeStruct((1, 1), jnp.float32),
            jax.ShapeDtypeStruct((1, 1), jnp.float32),
        ],
        scratch_shapes=[pltpu.SMEM((4,), jnp.float32)],
    )(h2, label_p, mask_p, vpad, bpad, upad, W0, W1, wsq)


def kernel(x, edge_index, label, mask, W0, W1, W_att, b_att, u_att):
    src = edge_index[:, 0, :].astype(jnp.int32).reshape(NMETA, NT, NEDGE // NT)
    dst = edge_index[:, 1, :].astype(jnp.int32).reshape(NMETA, NT, NEDGE // NT)
    pad = EPT - NEDGE // NT
    src = jnp.pad(src, ((0, 0), (0, 0), (0, pad)))
    dst = jnp.pad(dst, ((0, 0), (0, 0), (0, pad)), constant_values=N)
    src = src.reshape(NMETA, NT, NCHUNK, CH)
    dstp = dst.reshape(NMETA, NT, NCHUNK, CH)
    offs_c = (jnp.arange(NC, dtype=jnp.int32) * N).reshape(NC, 1, 1, 1, 1)
    srcp2 = src[None] + offs_c
    offs_m = (jnp.arange(NMETA, dtype=jnp.int32) * NACC).reshape(NMETA, 1, 1, 1)
    srcp3 = src + offs_m

    x2flat = jnp.concatenate([x[:, :HALF], x[:, HALF:]], axis=0)
    zrow = jnp.zeros((RPT, HALF), jnp.float32)

    g = _sc_pass1(x2flat, srcp2, dstp, zrow)

    pre2 = _tc_gcn(g, W0, W1)

    h2 = _sc_pass2(pre2.reshape(NMETA * NACC, D_OUT), srcp3, dstp, zrow)

    wat = W_att.reshape(N, D_OUT, NATT)
    wat = jnp.pad(wat, ((0, NACC - N), (0, 0), (0, 0)))
    wat = wat.transpose(2, 0, 1)
    v_sm, wsq = _tc_att(h2, wat)

    label_p = jnp.pad(label, ((0, NACC - N), (0, 0)))
    mask_p = jnp.pad(mask, ((0, NACC - N),)).reshape(NACC, 1)
    vpad = jnp.zeros((8, 128), jnp.float32).at[:NMETA, :NATT].set(v_sm)
    bpad = jnp.zeros((8, 128), jnp.float32).at[:NMETA, :NATT].set(
        jnp.broadcast_to(b_att, (NMETA, NATT)))
    upad = jnp.zeros((8, 128), jnp.float32).at[:, :NATT].set(
        jnp.broadcast_to(u_att, (8, NATT)))
    loss, acc = _tc_final(h2, label_p, mask_p, vpad, bpad, upad, W0, W1, wsq)
    return (loss[0, 0], acc[0, 0])
